# Initial kernel scaffold; baseline (speedup 1.0000x reference)
import jax
import jax.numpy as jnp
from jax.experimental import pallas as pl


def kernel(var_node_features, con_node_features, edge_features, rhs, params, edge_index, edge_types, assoc_con, assoc_var, node_types):
    raise NotImplementedError("write your pallas kernel here")


# speedup vs baseline: 42.6992x; 42.6992x over previous
import functools

import jax
import jax.numpy as jnp
from jax import lax
from jax.experimental import pallas as pl
from jax.experimental.pallas import tpu as pltpu
from jax.experimental.pallas import tpu_sc as plsc

DIM = 16
NV = 25000
N = 50000
E = 800000

NCORES = 2
NSUB = 16
NW = NCORES * NSUB
CHUNK = 128
WCH = 8
NCHUNK = -(-E // (NW * CHUNK * WCH)) * WCH
NWIN = NCHUNK // WCH
EW = NCHUNK * CHUNK
EPAD = NW * EW

AGG_ROWS = 50048
ROWS_PER_TILE = AGG_ROWS // NSUB
ZROWS = 391

BLK = 1000
NBLK = N // BLK
VBLK = NV // BLK


def _p16(w):
    out = jnp.zeros((DIM, DIM), jnp.float32)
    return out.at[: w.shape[0], : w.shape[1]].set(w)


def _pb16(b):
    return jnp.zeros((1, DIM), jnp.float32).at[0, : b.shape[0]].set(b)


def _conv_weights(cp):
    wh2 = jnp.zeros((DIM, DIM), jnp.float32).at[:15, 15].set(
        cp["hidden_to_var"]["l2"]["w"][:, 0])
    bh2 = jnp.zeros((1, DIM), jnp.float32).at[0, 15].set(
        cp["hidden_to_var"]["l2"]["b"][0])
    return (
        _p16(cp["mlp_cons"]["l1"]["w"]), _pb16(cp["mlp_cons"]["l1"]["b"]),
        _p16(cp["mlp_cons"]["l2"]["w"]), _pb16(cp["mlp_cons"]["l2"]["b"]),
        _p16(cp["mlp_vars"]["l1"]["w"]), _pb16(cp["mlp_vars"]["l1"]["b"]),
        _p16(cp["mlp_vars"]["l2"]["w"]), _pb16(cp["mlp_vars"]["l2"]["b"]),
        _p16(cp["hidden_to_var"]["l1"]["w"]), _pb16(cp["hidden_to_var"]["l1"]["b"]),
        wh2, bh2,
    )


_W16 = pl.BlockSpec((DIM, DIM), lambda i: (0, 0))
_B16 = pl.BlockSpec((1, DIM), lambda i: (0, 0))
_XBLK = pl.BlockSpec((BLK, DIM), lambda i: (i, 0))
_CBLK = pl.BlockSpec((BLK, 1), lambda i: (i, 0))
_TBLK = pl.BlockSpec((2, BLK, DIM), lambda i: (0, i, 0))
_CONV_W_SPECS = [_W16, _B16] * 6


def _dot(a, b):
    return jnp.dot(a, b, preferred_element_type=jnp.float32)


def _col_mask(k):
    c = lax.broadcasted_iota(jnp.int32, (1, DIM), 1)
    return (c == k).astype(jnp.float32)


def _tables(x, idg, ef, wc1, bc1, wc2, bc2, wv1, bv1, wv2, bv2, wh1, bh1,
            wh2, bh2):
    hc = jnp.maximum(_dot(x, wc1) + bc1, 0.0)
    mc = _dot(hc, wc2) + bc2
    hv = jnp.maximum(_dot(x, wv1) + bv1, 0.0)
    mv = _dot(hv, wv2) + bv2
    hh = jnp.maximum(_dot(x, wh1) + bh1, 0.0)
    va = _dot(hh, wh2) + bh2
    m15 = _col_mask(15)
    m0 = mc * idg + va * ef
    m1 = (mv + (x[:, 15:16] * ef) * m15) * idg
    return m0, m1


def _embed_body(f_ref, w1_ref, b1_ref, w2_ref, b2_ref, o_ref):
    f = f_ref[...]
    h = jnp.maximum(_dot(f, w1_ref[0]) + b1_ref[0], 0.0)
    y = _dot(h, w2_ref[0]) + b2_ref[0]
    y = y + f[:, 0:1] * _col_mask(13) + f[:, 1:2] * _col_mask(14)
    o_ref[...] = y


def _t1_body(x_ref, degp_ref, ef_ref, *refs):
    (wc1, bc1, wc2, bc2, wv1, bv1, wv2, bv2, wh1, bh1, wh2, bh2,
     t_ref, idg_ref) = refs
    deg = degp_ref[0, :, 0:1] + degp_ref[1, :, 0:1]
    idg = jnp.where(deg > 0, 1.0 / jnp.maximum(deg, 1.0), 0.0)
    idg_ref[...] = idg
    m0, m1 = _tables(x_ref[...], idg, ef_ref[...],
                     wc1[...], bc1[0], wc2[...], bc2[0],
                     wv1[...], bv1[0], wv2[...], bv2[0],
                     wh1[...], bh1[0], wh2[...], bh2[0])
    t_ref[0] = m0
    t_ref[1] = m1


def _post_epilogue(p_ref, rhs_ref, bias_ref):
    i = pl.program_id(0)
    aggr = p_ref[0] + p_ref[1]
    con = jnp.where(i >= VBLK, 1.0, 0.0)
    aggr = aggr - rhs_ref[...] * _col_mask(15) * con
    return jnp.maximum(aggr + bias_ref[0], 0.0)


def _post_body(p_ref, rhs_ref, bias_ref, idg_ref, ef_ref, *refs):
    (wc1, bc1, wc2, bc2, wv1, bv1, wv2, bv2, wh1, bh1, wh2, bh2,
     x_ref, t_ref) = refs
    x = _post_epilogue(p_ref, rhs_ref, bias_ref)
    x_ref[...] = x
    m0, m1 = _tables(x, idg_ref[...], ef_ref[...],
                     wc1[...], bc1[0], wc2[...], bc2[0],
                     wv1[...], bv1[0], wv2[...], bv2[0],
                     wh1[...], bh1[0], wh2[...], bh2[0])
    t_ref[0] = m0
    t_ref[1] = m1


def _post_final_body(p_ref, rhs_ref, bias_ref, x_ref):
    x_ref[...] = _post_epilogue(p_ref, rhs_ref, bias_ref)


def _head_body(*refs):
    xs = refs[0:7]
    w1, b1, w2, b2, w3, b3, w4, b4, o_ref = refs[7:]
    w1v = w1[...]
    acc = b1[0]
    for k in range(7):
        acc = acc + _dot(xs[k][...], w1v[16 * k:16 * (k + 1)])
    h = jnp.maximum(acc, 0.0)
    h = jnp.maximum(_dot(h, w2[...]) + b2[0], 0.0)
    h = jnp.maximum(_dot(h, w3[...]) + b3[0], 0.0)
    o_ref[...] = _dot(h, w4[...]) + b4[...]


def _sc_mesh():
    return plsc.VectorSubcoreMesh(core_axis_name="c", subcore_axis_name="s",
                                  num_cores=NCORES, num_subcores=NSUB)


_SC_PARAMS = pltpu.CompilerParams(use_tc_tiling_on_sc=False)


def _fill_rows(ref, nrows, value):
    def body(i, c):
        ref[i] = jnp.full((DIM,), value, jnp.float32)
        return c
    lax.fori_loop(0, nrows, body, 0)


def _zero_shared(shared, zbuf, sid):
    _fill_rows(zbuf, ZROWS, 0.0)
    base = sid * ROWS_PER_TILE
    for t in range(ROWS_PER_TILE // ZROWS):
        pltpu.sync_copy(zbuf, shared.at[pl.ds(base + t * ZROWS, ZROWS)])


def _copy_out(shared, out_hbm, ci, sid):
    base = sid * ROWS_PER_TILE
    pltpu.sync_copy(shared.at[pl.ds(base, ROWS_PER_TILE)],
                    out_hbm.at[ci, pl.ds(base, ROWS_PER_TILE)])


def _sc_degree(rowp):
    @functools.partial(
        pl.kernel,
        out_type=jax.ShapeDtypeStruct((NCORES, AGG_ROWS, DIM), jnp.float32),
        mesh=_sc_mesh(),
        compiler_params=_SC_PARAMS,
        scratch_types=[
            pltpu.VMEM_SHARED((AGG_ROWS, DIM), jnp.float32),
            pltpu.VMEM((WCH, CHUNK), jnp.int32),
            pltpu.VMEM((CHUNK, DIM), jnp.float32),
            pltpu.VMEM((ZROWS, DIM), jnp.float32),
        ],
    )
    def k(row_hbm, out_hbm, shared, idx_w, ones_v, zbuf):
        ci = lax.axis_index("c")
        sid = lax.axis_index("s")
        wid = sid * NCORES + ci
        _zero_shared(shared, zbuf, sid)
        _fill_rows(ones_v, CHUNK, 1.0)
        plsc.subcore_barrier()

        def win(g, c):
            off = pl.multiple_of(g * WCH, WCH)
            pltpu.sync_copy(row_hbm.at[wid, pl.ds(off, WCH)], idx_w)
            for b in range(WCH):
                pltpu.sync_copy(ones_v, shared.at[idx_w.at[b]], add=True)
            return c
        lax.fori_loop(0, NWIN, win, 0)
        plsc.subcore_barrier()
        _copy_out(shared, out_hbm, ci, sid)

    return k(rowp)


def _sc_aggregate(table, gidx, colp):
    @functools.partial(
        pl.kernel,
        out_type=jax.ShapeDtypeStruct((NCORES, AGG_ROWS, DIM), jnp.float32),
        mesh=_sc_mesh(),
        compiler_params=_SC_PARAMS,
        scratch_types=[
            pltpu.VMEM_SHARED((AGG_ROWS, DIM), jnp.float32),
            pltpu.VMEM((WCH, CHUNK), jnp.int32),
            pltpu.VMEM((WCH, CHUNK), jnp.int32),
            pltpu.VMEM((CHUNK, DIM), jnp.float32),
            pltpu.VMEM((ZROWS, DIM), jnp.float32),
            pltpu.SemaphoreType.DMA,
        ],
    )
    def k(t_hbm, gidx_hbm, col_hbm, out_hbm, shared, gidx_w, col_w, rows_v,
          zbuf, sem):
        ci = lax.axis_index("c")
        sid = lax.axis_index("s")
        wid = sid * NCORES + ci
        _zero_shared(shared, zbuf, sid)
        plsc.subcore_barrier()

        def win(g, c):
            off = pl.multiple_of(g * WCH, WCH)
            pltpu.sync_copy(gidx_hbm.at[wid, pl.ds(off, WCH)], gidx_w)
            pltpu.sync_copy(col_hbm.at[wid, pl.ds(off, WCH)], col_w)
            for b in range(WCH):
                pltpu.async_copy(t_hbm.at[gidx_w.at[b]], rows_v, sem).wait()
                pltpu.sync_copy(rows_v, shared.at[col_w.at[b]], add=True)
            return c
        lax.fori_loop(0, NWIN, win, 0)
        plsc.subcore_barrier()
        _copy_out(shared, out_hbm, ci, sid)

    return k(table, gidx, colp)


def _splitpad(a, fill):
    pad = jnp.full((EPAD - E,), fill, jnp.int32)
    return jnp.concatenate([a.astype(jnp.int32), pad]).reshape(NW, NCHUNK, CHUNK)


def kernel(var_node_features, con_node_features, edge_features, rhs, params,
           edge_index, edge_types, assoc_con, assoc_var, node_types):
    row = edge_index[0]
    col = edge_index[1]

    gidx = _splitpad(row + N * edge_types, 0)
    colp = _splitpad(col, N)
    rowp = _splitpad(row, N)
    ef = edge_features
    rhs2d = rhs[:, None]
    feats = jnp.concatenate([var_node_features, con_node_features], axis=0)
    feats = jnp.pad(feats, ((0, 0), (0, DIM - feats.shape[1])))
    ew1 = jnp.stack([_p16(params["var_mlp"]["l1"]["w"]),
                     _p16(params["con_mlp"]["l1"]["w"])])
    eb1 = jnp.stack([_pb16(params["var_mlp"]["l1"]["b"]),
                     _pb16(params["con_mlp"]["l1"]["b"])])
    ew2 = jnp.stack([_p16(params["var_mlp"]["l2"]["w"]),
                     _p16(params["con_mlp"]["l2"]["w"])])
    eb2 = jnp.stack([_pb16(params["var_mlp"]["l2"]["b"]),
                     _pb16(params["con_mlp"]["l2"]["b"])])
    convw = [_conv_weights(cp) for cp in params["convs"]]
    biases = [cp["bias"][None, :] for cp in params["convs"]]

    xsh = jax.ShapeDtypeStruct((N, DIM), jnp.float32)
    tsh = jax.ShapeDtypeStruct((2, N, DIM), jnp.float32)
    csh = jax.ShapeDtypeStruct((N, 1), jnp.float32)

    emb_spec = pl.BlockSpec((1, DIM, DIM), lambda i: (i // VBLK, 0, 0))
    emb_bspec = pl.BlockSpec((1, 1, DIM), lambda i: (i // VBLK, 0, 0))
    x0 = pl.pallas_call(
        _embed_body, grid=(NBLK,),
        in_specs=[_XBLK, emb_spec, emb_bspec, emb_spec, emb_bspec],
        out_specs=_XBLK, out_shape=xsh,
    )(feats, ew1, eb1, ew2, eb2)

    degp = _sc_degree(rowp)

    pspec = pl.BlockSpec((2, BLK, DIM), lambda i: (0, i, 0))
    t1, inv_deg = pl.pallas_call(
        _t1_body, grid=(NBLK,),
        in_specs=[_XBLK, pspec, _CBLK] + _CONV_W_SPECS,
        out_specs=(_TBLK, _CBLK), out_shape=(tsh, csh),
    )(x0, degp, ef, *convw[0])

    rhs_spec = pl.BlockSpec((BLK, 1), lambda i: (jnp.maximum(i - VBLK, 0), 0))
    xs = [x0]
    t = t1
    for layer in range(6):
        partial = _sc_aggregate(t.reshape(2 * N, DIM), gidx, colp)
        if layer < 5:
            x, t = pl.pallas_call(
                _post_body, grid=(NBLK,),
                in_specs=[pspec, rhs_spec, _B16, _CBLK, _CBLK] + _CONV_W_SPECS,
                out_specs=(_XBLK, _TBLK), out_shape=(xsh, tsh),
            )(partial, rhs2d, biases[layer], inv_deg, ef, *convw[layer + 1])
        else:
            x = pl.pallas_call(
                _post_final_body, grid=(NBLK,),
                in_specs=[pspec, rhs_spec, _B16],
                out_specs=_XBLK, out_shape=xsh,
            )(partial, rhs2d, biases[layer])
        xs.append(x)

    vspec = pl.BlockSpec((BLK, DIM), lambda i: (i, 0))
    w1spec = pl.BlockSpec((7 * DIM, DIM), lambda i: (0, 0))
    w4spec = pl.BlockSpec((DIM, 1), lambda i: (0, 0))
    b4spec = pl.BlockSpec((1, 1), lambda i: (0, 0))
    out = pl.pallas_call(
        _head_body, grid=(VBLK,),
        in_specs=[vspec] * 7 + [w1spec, _B16, _W16, _B16, _W16, _B16,
                                w4spec, b4spec],
        out_specs=pl.BlockSpec((BLK, 1), lambda i: (i, 0)),
        out_shape=jax.ShapeDtypeStruct((NV, 1), jnp.float32),
    )(*xs,
      params["fc1"]["w"], params["fc1"]["b"][None, :],
      params["fc2"]["w"], params["fc2"]["b"][None, :],
      params["fc3"]["w"], params["fc3"]["b"][None, :],
      params["fc4"]["w"], params["fc4"]["b"][None, :])
    return out[:, 0]


# speedup vs baseline: 54.7522x; 1.2823x over previous
import functools

import jax
import jax.numpy as jnp
from jax import lax
from jax.experimental import pallas as pl
from jax.experimental.pallas import tpu as pltpu
from jax.experimental.pallas import tpu_sc as plsc

DIM = 16
NV = 25000
N = 50000
E = 800000

NCORES = 2
NSUB = 16
NW = NCORES * NSUB
CHUNK = 128
NB = 8
NCHUNK = -(-E // (NW * CHUNK * NB)) * NB
NBATCH = NCHUNK // NB
NPAIR = NBATCH // 2
LASTB = (NBATCH - 1) * NB
EW = NCHUNK * CHUNK
EPAD = NW * EW

AGG_ROWS = 50048
ROWS_PER_TILE = AGG_ROWS // NSUB
ZROWS = 391

BLK = 1000
NBLK = N // BLK
VBLK = NV // BLK


def _p16(w):
    out = jnp.zeros((DIM, DIM), jnp.float32)
    return out.at[: w.shape[0], : w.shape[1]].set(w)


def _pb16(b):
    return jnp.zeros((1, DIM), jnp.float32).at[0, : b.shape[0]].set(b)


def _conv_weights(cp):
    wh2 = jnp.zeros((DIM, DIM), jnp.float32).at[:15, 15].set(
        cp["hidden_to_var"]["l2"]["w"][:, 0])
    bh2 = jnp.zeros((1, DIM), jnp.float32).at[0, 15].set(
        cp["hidden_to_var"]["l2"]["b"][0])
    return (
        _p16(cp["mlp_cons"]["l1"]["w"]), _pb16(cp["mlp_cons"]["l1"]["b"]),
        _p16(cp["mlp_cons"]["l2"]["w"]), _pb16(cp["mlp_cons"]["l2"]["b"]),
        _p16(cp["mlp_vars"]["l1"]["w"]), _pb16(cp["mlp_vars"]["l1"]["b"]),
        _p16(cp["mlp_vars"]["l2"]["w"]), _pb16(cp["mlp_vars"]["l2"]["b"]),
        _p16(cp["hidden_to_var"]["l1"]["w"]), _pb16(cp["hidden_to_var"]["l1"]["b"]),
        wh2, bh2,
    )


_W16 = pl.BlockSpec((DIM, DIM), lambda i: (0, 0))
_B16 = pl.BlockSpec((1, DIM), lambda i: (0, 0))
_XBLK = pl.BlockSpec((BLK, DIM), lambda i: (i, 0))
_CBLK = pl.BlockSpec((BLK, 1), lambda i: (i, 0))
_TBLK = pl.BlockSpec((2, BLK, DIM), lambda i: (0, i, 0))
_CONV_W_SPECS = [_W16, _B16] * 6


def _dot(a, b):
    return jnp.dot(a, b, preferred_element_type=jnp.float32)


def _col_mask(k):
    c = lax.broadcasted_iota(jnp.int32, (1, DIM), 1)
    return (c == k).astype(jnp.float32)


def _tables(x, idg, ef, wc1, bc1, wc2, bc2, wv1, bv1, wv2, bv2, wh1, bh1,
            wh2, bh2):
    hc = jnp.maximum(_dot(x, wc1) + bc1, 0.0)
    mc = _dot(hc, wc2) + bc2
    hv = jnp.maximum(_dot(x, wv1) + bv1, 0.0)
    mv = _dot(hv, wv2) + bv2
    hh = jnp.maximum(_dot(x, wh1) + bh1, 0.0)
    va = _dot(hh, wh2) + bh2
    m15 = _col_mask(15)
    m0 = mc * idg + va * ef
    m1 = (mv + (x[:, 15:16] * ef) * m15) * idg
    return m0, m1


def _embed_body(f_ref, w1_ref, b1_ref, w2_ref, b2_ref, o_ref):
    f = f_ref[...]
    h = jnp.maximum(_dot(f, w1_ref[0]) + b1_ref[0], 0.0)
    y = _dot(h, w2_ref[0]) + b2_ref[0]
    y = y + f[:, 0:1] * _col_mask(13) + f[:, 1:2] * _col_mask(14)
    o_ref[...] = y


def _t1_body(x_ref, degp_ref, ef_ref, *refs):
    (wc1, bc1, wc2, bc2, wv1, bv1, wv2, bv2, wh1, bh1, wh2, bh2,
     t_ref, idg_ref) = refs
    deg = degp_ref[0, :, 0:1] + degp_ref[1, :, 0:1]
    idg = jnp.where(deg > 0, 1.0 / jnp.maximum(deg, 1.0), 0.0)
    idg_ref[...] = idg
    m0, m1 = _tables(x_ref[...], idg, ef_ref[...],
                     wc1[...], bc1[0], wc2[...], bc2[0],
                     wv1[...], bv1[0], wv2[...], bv2[0],
                     wh1[...], bh1[0], wh2[...], bh2[0])
    t_ref[0] = m0
    t_ref[1] = m1


def _post_epilogue(p_ref, rhs_ref, bias_ref):
    i = pl.program_id(0)
    aggr = p_ref[0] + p_ref[1]
    con = jnp.where(i >= VBLK, 1.0, 0.0)
    aggr = aggr - rhs_ref[...] * _col_mask(15) * con
    return jnp.maximum(aggr + bias_ref[0], 0.0)


def _post_body(p_ref, rhs_ref, bias_ref, idg_ref, ef_ref, *refs):
    (wc1, bc1, wc2, bc2, wv1, bv1, wv2, bv2, wh1, bh1, wh2, bh2,
     x_ref, t_ref) = refs
    x = _post_epilogue(p_ref, rhs_ref, bias_ref)
    x_ref[...] = x
    m0, m1 = _tables(x, idg_ref[...], ef_ref[...],
                     wc1[...], bc1[0], wc2[...], bc2[0],
                     wv1[...], bv1[0], wv2[...], bv2[0],
                     wh1[...], bh1[0], wh2[...], bh2[0])
    t_ref[0] = m0
    t_ref[1] = m1


def _post_final_body(p_ref, rhs_ref, bias_ref, x_ref):
    x_ref[...] = _post_epilogue(p_ref, rhs_ref, bias_ref)


def _head_body(*refs):
    xs = refs[0:7]
    w1, b1, w2, b2, w3, b3, w4, b4, o_ref = refs[7:]
    w1v = w1[...]
    acc = b1[0]
    for k in range(7):
        acc = acc + _dot(xs[k][...], w1v[16 * k:16 * (k + 1)])
    h = jnp.maximum(acc, 0.0)
    h = jnp.maximum(_dot(h, w2[...]) + b2[0], 0.0)
    h = jnp.maximum(_dot(h, w3[...]) + b3[0], 0.0)
    o_ref[...] = _dot(h, w4[...]) + b4[...]


def _sc_mesh():
    return plsc.VectorSubcoreMesh(core_axis_name="c", subcore_axis_name="s",
                                  num_cores=NCORES, num_subcores=NSUB)


_SC_PARAMS = pltpu.CompilerParams(use_tc_tiling_on_sc=False)


def _fill_rows(ref, nrows, value):
    def body(i, c):
        ref[i] = jnp.full((DIM,), value, jnp.float32)
        return c
    lax.fori_loop(0, nrows, body, 0)


def _zero_shared_async(shared, zbuf, sid, zsem):
    _fill_rows(zbuf, ZROWS, 0.0)
    base = sid * ROWS_PER_TILE
    return [pltpu.async_copy(zbuf, shared.at[pl.ds(base + t * ZROWS, ZROWS)],
                             zsem)
            for t in range(ROWS_PER_TILE // ZROWS)]


def _copy_out(shared, out_hbm, ci, sid):
    base = sid * ROWS_PER_TILE
    pltpu.sync_copy(shared.at[pl.ds(base, ROWS_PER_TILE)],
                    out_hbm.at[ci, pl.ds(base, ROWS_PER_TILE)])


def _batch_offsets(kk):
    offa = pl.multiple_of(jnp.minimum((2 * kk + 2) * NB, LASTB), NB)
    offb = pl.multiple_of(jnp.minimum((2 * kk + 3) * NB, LASTB), NB)
    return offa, offb


def _sc_degree(rowp):
    @functools.partial(
        pl.kernel,
        out_type=jax.ShapeDtypeStruct((NCORES, AGG_ROWS, DIM), jnp.float32),
        mesh=_sc_mesh(),
        compiler_params=_SC_PARAMS,
        scratch_types=[
            pltpu.VMEM_SHARED((AGG_ROWS, DIM), jnp.float32),
            pltpu.VMEM((2, NB, CHUNK), jnp.int32),
            pltpu.VMEM((CHUNK, DIM), jnp.float32),
            pltpu.VMEM((ZROWS, DIM), jnp.float32),
            pltpu.SemaphoreType.DMA,
            pltpu.SemaphoreType.DMA,
            pltpu.SemaphoreType.DMA,
            pltpu.SemaphoreType.DMA,
            pltpu.SemaphoreType.DMA,
        ],
    )
    def k(row_hbm, out_hbm, shared, idx_w, ones_v, zbuf,
          ssem0, ssem1, isem0, isem1, zsem):
        ci = lax.axis_index("c")
        sid = lax.axis_index("s")
        wid = sid * NCORES + ci
        zcps = _zero_shared_async(shared, zbuf, sid, zsem)
        pltpu.sync_copy(row_hbm.at[wid, pl.ds(0, NB)], idx_w.at[0])
        pltpu.sync_copy(row_hbm.at[wid, pl.ds(NB, NB)], idx_w.at[1])
        _fill_rows(ones_v, CHUNK, 1.0)
        for cp in zcps:
            cp.wait()
        plsc.subcore_barrier()

        def fire(slot, sem):
            return [pltpu.async_copy(ones_v, shared.at[idx_w.at[slot, j]],
                                     sem, add=True)
                    for j in range(NB)]

        def body(kk, c):
            spa = fire(0, ssem0)
            spb = fire(1, ssem1)
            offa, offb = _batch_offsets(kk)
            for cp in spa:
                cp.wait()
            ia = pltpu.async_copy(row_hbm.at[wid, pl.ds(offa, NB)],
                                  idx_w.at[0], isem0)
            for cp in spb:
                cp.wait()
            ib = pltpu.async_copy(row_hbm.at[wid, pl.ds(offb, NB)],
                                  idx_w.at[1], isem1)
            ia.wait()
            ib.wait()
            return c
        lax.fori_loop(0, NPAIR, body, 0)

        for cp in fire(0, ssem0):
            cp.wait()
        plsc.subcore_barrier()
        _copy_out(shared, out_hbm, ci, sid)

    return k(rowp)


def _sc_aggregate(table, gidx, colp):
    @functools.partial(
        pl.kernel,
        out_type=jax.ShapeDtypeStruct((NCORES, AGG_ROWS, DIM), jnp.float32),
        mesh=_sc_mesh(),
        compiler_params=_SC_PARAMS,
        scratch_types=[
            pltpu.VMEM_SHARED((AGG_ROWS, DIM), jnp.float32),
            pltpu.VMEM((2, NB, CHUNK), jnp.int32),
            pltpu.VMEM((2, NB, CHUNK), jnp.int32),
            pltpu.VMEM((2, NB, CHUNK, DIM), jnp.float32),
            pltpu.VMEM((ZROWS, DIM), jnp.float32),
            pltpu.SemaphoreType.DMA,
            pltpu.SemaphoreType.DMA,
            pltpu.SemaphoreType.DMA,
            pltpu.SemaphoreType.DMA,
            pltpu.SemaphoreType.DMA,
            pltpu.SemaphoreType.DMA,
            pltpu.SemaphoreType.DMA,
        ],
    )
    def k(t_hbm, gidx_hbm, col_hbm, out_hbm, shared, gidx_w, col_w, rows,
          zbuf, gsem0, gsem1, ssem0, ssem1, isem0, isem1, zsem):
        ci = lax.axis_index("c")
        sid = lax.axis_index("s")
        wid = sid * NCORES + ci
        gsems = (gsem0, gsem1)
        ssems = (ssem0, ssem1)
        zcps = _zero_shared_async(shared, zbuf, sid, zsem)
        pltpu.sync_copy(gidx_hbm.at[wid, pl.ds(0, NB)], gidx_w.at[0])
        pltpu.sync_copy(col_hbm.at[wid, pl.ds(0, NB)], col_w.at[0])
        pltpu.sync_copy(gidx_hbm.at[wid, pl.ds(NB, NB)], gidx_w.at[1])
        pltpu.sync_copy(col_hbm.at[wid, pl.ds(NB, NB)], col_w.at[1])
        for cp in zcps:
            cp.wait()
        plsc.subcore_barrier()

        def fire_gathers(slot):
            for j in range(NB):
                pltpu.async_copy(t_hbm.at[gidx_w.at[slot, j]],
                                 rows.at[slot, j], gsems[slot])

        def drain_gathers(slot):
            for j in range(NB):
                pltpu.make_async_copy(t_hbm.at[gidx_w.at[slot, j]],
                                      rows.at[slot, j], gsems[slot]).wait()

        def fire_scatters(slot):
            return [pltpu.async_copy(rows.at[slot, j],
                                     shared.at[col_w.at[slot, j]],
                                     ssems[slot], add=True)
                    for j in range(NB)]

        fire_gathers(0)
        fire_gathers(1)

        def body(kk, c):
            drain_gathers(0)
            spa = fire_scatters(0)
            drain_gathers(1)
            spb = fire_scatters(1)
            offa, offb = _batch_offsets(kk)
            for cp in spa:
                cp.wait()
            ia = [pltpu.async_copy(gidx_hbm.at[wid, pl.ds(offa, NB)],
                                   gidx_w.at[0], isem0),
                  pltpu.async_copy(col_hbm.at[wid, pl.ds(offa, NB)],
                                   col_w.at[0], isem0)]
            for cp in spb:
                cp.wait()
            ib = [pltpu.async_copy(gidx_hbm.at[wid, pl.ds(offb, NB)],
                                   gidx_w.at[1], isem1),
                  pltpu.async_copy(col_hbm.at[wid, pl.ds(offb, NB)],
                                   col_w.at[1], isem1)]
            for cp in ia:
                cp.wait()
            fire_gathers(0)
            for cp in ib:
                cp.wait()
            fire_gathers(1)
            return c
        lax.fori_loop(0, NPAIR, body, 0)

        drain_gathers(0)
        spa = fire_scatters(0)
        drain_gathers(1)
        for cp in spa:
            cp.wait()
        plsc.subcore_barrier()
        _copy_out(shared, out_hbm, ci, sid)

    return k(table, gidx, colp)


def _splitpad(a, fill):
    pad = jnp.full((EPAD - E,), fill, jnp.int32)
    return jnp.concatenate([a.astype(jnp.int32), pad]).reshape(NW, NCHUNK, CHUNK)


def kernel(var_node_features, con_node_features, edge_features, rhs, params,
           edge_index, edge_types, assoc_con, assoc_var, node_types):
    row = edge_index[0]
    col = edge_index[1]

    gidx = _splitpad(row + N * edge_types, 0)
    colp = _splitpad(col, N)
    rowp = _splitpad(row, N)
    ef = edge_features
    rhs2d = rhs[:, None]
    feats = jnp.concatenate([var_node_features, con_node_features], axis=0)
    feats = jnp.pad(feats, ((0, 0), (0, DIM - feats.shape[1])))
    ew1 = jnp.stack([_p16(params["var_mlp"]["l1"]["w"]),
                     _p16(params["con_mlp"]["l1"]["w"])])
    eb1 = jnp.stack([_pb16(params["var_mlp"]["l1"]["b"]),
                     _pb16(params["con_mlp"]["l1"]["b"])])
    ew2 = jnp.stack([_p16(params["var_mlp"]["l2"]["w"]),
                     _p16(params["con_mlp"]["l2"]["w"])])
    eb2 = jnp.stack([_pb16(params["var_mlp"]["l2"]["b"]),
                     _pb16(params["con_mlp"]["l2"]["b"])])
    convw = [_conv_weights(cp) for cp in params["convs"]]
    biases = [cp["bias"][None, :] for cp in params["convs"]]

    xsh = jax.ShapeDtypeStruct((N, DIM), jnp.float32)
    tsh = jax.ShapeDtypeStruct((2, N, DIM), jnp.float32)
    csh = jax.ShapeDtypeStruct((N, 1), jnp.float32)

    emb_spec = pl.BlockSpec((1, DIM, DIM), lambda i: (i // VBLK, 0, 0))
    emb_bspec = pl.BlockSpec((1, 1, DIM), lambda i: (i // VBLK, 0, 0))
    x0 = pl.pallas_call(
        _embed_body, grid=(NBLK,),
        in_specs=[_XBLK, emb_spec, emb_bspec, emb_spec, emb_bspec],
        out_specs=_XBLK, out_shape=xsh,
    )(feats, ew1, eb1, ew2, eb2)

    degp = _sc_degree(rowp)

    pspec = pl.BlockSpec((2, BLK, DIM), lambda i: (0, i, 0))
    t1, inv_deg = pl.pallas_call(
        _t1_body, grid=(NBLK,),
        in_specs=[_XBLK, pspec, _CBLK] + _CONV_W_SPECS,
        out_specs=(_TBLK, _CBLK), out_shape=(tsh, csh),
    )(x0, degp, ef, *convw[0])

    rhs_spec = pl.BlockSpec((BLK, 1), lambda i: (jnp.maximum(i - VBLK, 0), 0))
    xs = [x0]
    t = t1
    for layer in range(6):
        partial = _sc_aggregate(t.reshape(2 * N, DIM), gidx, colp)
        if layer < 5:
            x, t = pl.pallas_call(
                _post_body, grid=(NBLK,),
                in_specs=[pspec, rhs_spec, _B16, _CBLK, _CBLK] + _CONV_W_SPECS,
                out_specs=(_XBLK, _TBLK), out_shape=(xsh, tsh),
            )(partial, rhs2d, biases[layer], inv_deg, ef, *convw[layer + 1])
        else:
            x = pl.pallas_call(
                _post_final_body, grid=(NBLK,),
                in_specs=[pspec, rhs_spec, _B16],
                out_specs=_XBLK, out_shape=xsh,
            )(partial, rhs2d, biases[layer])
        xs.append(x)

    vspec = pl.BlockSpec((BLK, DIM), lambda i: (i, 0))
    w1spec = pl.BlockSpec((7 * DIM, DIM), lambda i: (0, 0))
    w4spec = pl.BlockSpec((DIM, 1), lambda i: (0, 0))
    b4spec = pl.BlockSpec((1, 1), lambda i: (0, 0))
    out = pl.pallas_call(
        _head_body, grid=(VBLK,),
        in_specs=[vspec] * 7 + [w1spec, _B16, _W16, _B16, _W16, _B16,
                                w4spec, b4spec],
        out_specs=pl.BlockSpec((BLK, 1), lambda i: (i, 0)),
        out_shape=jax.ShapeDtypeStruct((NV, 1), jnp.float32),
    )(*xs,
      params["fc1"]["w"], params["fc1"]["b"][None, :],
      params["fc2"]["w"], params["fc2"]["b"][None, :],
      params["fc3"]["w"], params["fc3"]["b"][None, :],
      params["fc4"]["w"], params["fc4"]["b"][None, :])
    return out[:, 0]


# speedup vs baseline: 59.1902x; 1.0811x over previous
import functools

import jax
import jax.numpy as jnp
from jax import lax
from jax.experimental import pallas as pl
from jax.experimental.pallas import tpu as pltpu
from jax.experimental.pallas import tpu_sc as plsc

DIM = 16
NV = 25000
N = 50000
E = 800000

NCORES = 2
NSUB = 16
NW = NCORES * NSUB
CHUNK = 128
NB = 8
NCHUNK = -(-E // (NW * CHUNK * NB)) * NB
NBATCH = NCHUNK // NB
NPAIR = NBATCH // 2
LASTB = (NBATCH - 1) * NB
EW = NCHUNK * CHUNK
EPAD = NW * EW

AGG_ROWS = 50048
ROWS_PER_TILE = AGG_ROWS // NSUB
ZROWS = 391

BLK = 5000
NBLK = N // BLK
VBLK = NV // BLK


def _p16(w):
    out = jnp.zeros((DIM, DIM), jnp.float32)
    return out.at[: w.shape[0], : w.shape[1]].set(w)


def _pb16(b):
    return jnp.zeros((1, DIM), jnp.float32).at[0, : b.shape[0]].set(b)


def _conv_weights(cp):
    wh2 = jnp.zeros((DIM, DIM), jnp.float32).at[:15, 15].set(
        cp["hidden_to_var"]["l2"]["w"][:, 0])
    bh2 = jnp.zeros((1, DIM), jnp.float32).at[0, 15].set(
        cp["hidden_to_var"]["l2"]["b"][0])
    return (
        _p16(cp["mlp_cons"]["l1"]["w"]), _pb16(cp["mlp_cons"]["l1"]["b"]),
        _p16(cp["mlp_cons"]["l2"]["w"]), _pb16(cp["mlp_cons"]["l2"]["b"]),
        _p16(cp["mlp_vars"]["l1"]["w"]), _pb16(cp["mlp_vars"]["l1"]["b"]),
        _p16(cp["mlp_vars"]["l2"]["w"]), _pb16(cp["mlp_vars"]["l2"]["b"]),
        _p16(cp["hidden_to_var"]["l1"]["w"]), _pb16(cp["hidden_to_var"]["l1"]["b"]),
        wh2, bh2,
    )


_W16 = pl.BlockSpec((DIM, DIM), lambda i: (0, 0))
_B16 = pl.BlockSpec((1, DIM), lambda i: (0, 0))
_XBLK = pl.BlockSpec((BLK, DIM), lambda i: (i, 0))
_CBLK = pl.BlockSpec((BLK, 1), lambda i: (i, 0))
_TBLK = pl.BlockSpec((2, BLK, DIM), lambda i: (0, i, 0))
_CONV_W_SPECS = [_W16, _B16] * 6


def _dot(a, b):
    return jnp.dot(a, b, preferred_element_type=jnp.float32)


def _col_mask(k):
    c = lax.broadcasted_iota(jnp.int32, (1, DIM), 1)
    return (c == k).astype(jnp.float32)


def _tables(x, idg, ef, wc1, bc1, wc2, bc2, wv1, bv1, wv2, bv2, wh1, bh1,
            wh2, bh2):
    hc = jnp.maximum(_dot(x, wc1) + bc1, 0.0)
    mc = _dot(hc, wc2) + bc2
    hv = jnp.maximum(_dot(x, wv1) + bv1, 0.0)
    mv = _dot(hv, wv2) + bv2
    hh = jnp.maximum(_dot(x, wh1) + bh1, 0.0)
    va = _dot(hh, wh2) + bh2
    m15 = _col_mask(15)
    m0 = mc * idg + va * ef
    m1 = (mv + (x[:, 15:16] * ef) * m15) * idg
    return m0, m1


def _embed_body(f_ref, w1_ref, b1_ref, w2_ref, b2_ref, o_ref):
    f = f_ref[...]
    h = jnp.maximum(_dot(f, w1_ref[0]) + b1_ref[0], 0.0)
    y = _dot(h, w2_ref[0]) + b2_ref[0]
    y = y + f[:, 0:1] * _col_mask(13) + f[:, 1:2] * _col_mask(14)
    o_ref[...] = y


def _t1_body(x_ref, degp_ref, ef_ref, *refs):
    (wc1, bc1, wc2, bc2, wv1, bv1, wv2, bv2, wh1, bh1, wh2, bh2,
     t_ref, idg_ref) = refs
    deg = degp_ref[0, :, 0:1] + degp_ref[1, :, 0:1]
    idg = jnp.where(deg > 0, 1.0 / jnp.maximum(deg, 1.0), 0.0)
    idg_ref[...] = idg
    m0, m1 = _tables(x_ref[...], idg, ef_ref[...],
                     wc1[...], bc1[0], wc2[...], bc2[0],
                     wv1[...], bv1[0], wv2[...], bv2[0],
                     wh1[...], bh1[0], wh2[...], bh2[0])
    t_ref[0] = m0
    t_ref[1] = m1


def _post_epilogue(p_ref, rhs_ref, bias_ref):
    i = pl.program_id(0)
    aggr = p_ref[0] + p_ref[1]
    con = jnp.where(i >= VBLK, 1.0, 0.0)
    aggr = aggr - rhs_ref[...] * _col_mask(15) * con
    return jnp.maximum(aggr + bias_ref[0], 0.0)


def _post_body(p_ref, rhs_ref, bias_ref, idg_ref, ef_ref, *refs):
    (wc1, bc1, wc2, bc2, wv1, bv1, wv2, bv2, wh1, bh1, wh2, bh2,
     x_ref, t_ref) = refs
    x = _post_epilogue(p_ref, rhs_ref, bias_ref)
    x_ref[...] = x
    m0, m1 = _tables(x, idg_ref[...], ef_ref[...],
                     wc1[...], bc1[0], wc2[...], bc2[0],
                     wv1[...], bv1[0], wv2[...], bv2[0],
                     wh1[...], bh1[0], wh2[...], bh2[0])
    t_ref[0] = m0
    t_ref[1] = m1


def _post_final_body(p_ref, rhs_ref, bias_ref, x_ref):
    x_ref[...] = _post_epilogue(p_ref, rhs_ref, bias_ref)


def _head_body(*refs):
    xs = refs[0:7]
    w1, b1, w2, b2, w3, b3, w4, b4, o_ref = refs[7:]
    w1v = w1[...]
    acc = b1[0]
    for k in range(7):
        acc = acc + _dot(xs[k][...], w1v[16 * k:16 * (k + 1)])
    h = jnp.maximum(acc, 0.0)
    h = jnp.maximum(_dot(h, w2[...]) + b2[0], 0.0)
    h = jnp.maximum(_dot(h, w3[...]) + b3[0], 0.0)
    o_ref[...] = _dot(h, w4[...]) + b4[...]


def _sc_mesh():
    return plsc.VectorSubcoreMesh(core_axis_name="c", subcore_axis_name="s",
                                  num_cores=NCORES, num_subcores=NSUB)


_SC_PARAMS = pltpu.CompilerParams(use_tc_tiling_on_sc=False)


def _fill_rows(ref, nrows, value):
    def body(i, c):
        ref[i] = jnp.full((DIM,), value, jnp.float32)
        return c
    lax.fori_loop(0, nrows, body, 0)


def _zero_shared_async(shared, zbuf, sid, zsem):
    _fill_rows(zbuf, ZROWS, 0.0)
    base = sid * ROWS_PER_TILE
    return [pltpu.async_copy(zbuf, shared.at[pl.ds(base + t * ZROWS, ZROWS)],
                             zsem)
            for t in range(ROWS_PER_TILE // ZROWS)]


def _copy_out(shared, out_hbm, ci, sid):
    base = sid * ROWS_PER_TILE
    pltpu.sync_copy(shared.at[pl.ds(base, ROWS_PER_TILE)],
                    out_hbm.at[ci, pl.ds(base, ROWS_PER_TILE)])


def _batch_offsets(kk):
    offa = pl.multiple_of(jnp.minimum((2 * kk + 2) * NB, LASTB), NB)
    offb = pl.multiple_of(jnp.minimum((2 * kk + 3) * NB, LASTB), NB)
    return offa, offb


def _sc_degree(rowp):
    @functools.partial(
        pl.kernel,
        out_type=jax.ShapeDtypeStruct((NCORES, AGG_ROWS, DIM), jnp.float32),
        mesh=_sc_mesh(),
        compiler_params=_SC_PARAMS,
        scratch_types=[
            pltpu.VMEM_SHARED((AGG_ROWS, DIM), jnp.float32),
            pltpu.VMEM((2, NB, CHUNK), jnp.int32),
            pltpu.VMEM((CHUNK, DIM), jnp.float32),
            pltpu.VMEM((ZROWS, DIM), jnp.float32),
            pltpu.SemaphoreType.DMA,
            pltpu.SemaphoreType.DMA,
            pltpu.SemaphoreType.DMA,
            pltpu.SemaphoreType.DMA,
            pltpu.SemaphoreType.DMA,
        ],
    )
    def k(row_hbm, out_hbm, shared, idx_w, ones_v, zbuf,
          ssem0, ssem1, isem0, isem1, zsem):
        ci = lax.axis_index("c")
        sid = lax.axis_index("s")
        wid = sid * NCORES + ci
        zcps = _zero_shared_async(shared, zbuf, sid, zsem)
        pltpu.sync_copy(row_hbm.at[wid, pl.ds(0, NB)], idx_w.at[0])
        pltpu.sync_copy(row_hbm.at[wid, pl.ds(NB, NB)], idx_w.at[1])
        _fill_rows(ones_v, CHUNK, 1.0)
        for cp in zcps:
            cp.wait()
        plsc.subcore_barrier()

        def fire(slot, sem):
            return [pltpu.async_copy(ones_v, shared.at[idx_w.at[slot, j]],
                                     sem, add=True)
                    for j in range(NB)]

        def body(kk, c):
            spa = fire(0, ssem0)
            spb = fire(1, ssem1)
            offa, offb = _batch_offsets(kk)
            for cp in spa:
                cp.wait()
            ia = pltpu.async_copy(row_hbm.at[wid, pl.ds(offa, NB)],
                                  idx_w.at[0], isem0)
            for cp in spb:
                cp.wait()
            ib = pltpu.async_copy(row_hbm.at[wid, pl.ds(offb, NB)],
                                  idx_w.at[1], isem1)
            ia.wait()
            ib.wait()
            return c
        lax.fori_loop(0, NPAIR, body, 0)

        for cp in fire(0, ssem0):
            cp.wait()
        plsc.subcore_barrier()
        _copy_out(shared, out_hbm, ci, sid)

    return k(rowp)


def _sc_aggregate(table, gidx, colp):
    @functools.partial(
        pl.kernel,
        out_type=jax.ShapeDtypeStruct((NCORES, AGG_ROWS, DIM), jnp.float32),
        mesh=_sc_mesh(),
        compiler_params=_SC_PARAMS,
        scratch_types=[
            pltpu.VMEM_SHARED((AGG_ROWS, DIM), jnp.float32),
            pltpu.VMEM((2, NB, CHUNK), jnp.int32),
            pltpu.VMEM((2, NB, CHUNK), jnp.int32),
            pltpu.VMEM((2, NB, CHUNK, DIM), jnp.float32),
            pltpu.VMEM((ZROWS, DIM), jnp.float32),
            pltpu.SemaphoreType.DMA,
            pltpu.SemaphoreType.DMA,
            pltpu.SemaphoreType.DMA,
            pltpu.SemaphoreType.DMA,
            pltpu.SemaphoreType.DMA,
            pltpu.SemaphoreType.DMA,
            pltpu.SemaphoreType.DMA,
        ],
    )
    def k(t_hbm, gidx_hbm, col_hbm, out_hbm, shared, gidx_w, col_w, rows,
          zbuf, gsem0, gsem1, ssem0, ssem1, isem0, isem1, zsem):
        ci = lax.axis_index("c")
        sid = lax.axis_index("s")
        wid = sid * NCORES + ci
        gsems = (gsem0, gsem1)
        ssems = (ssem0, ssem1)
        zcps = _zero_shared_async(shared, zbuf, sid, zsem)
        pltpu.sync_copy(gidx_hbm.at[wid, pl.ds(0, NB)], gidx_w.at[0])
        pltpu.sync_copy(col_hbm.at[wid, pl.ds(0, NB)], col_w.at[0])
        pltpu.sync_copy(gidx_hbm.at[wid, pl.ds(NB, NB)], gidx_w.at[1])
        pltpu.sync_copy(col_hbm.at[wid, pl.ds(NB, NB)], col_w.at[1])
        for cp in zcps:
            cp.wait()
        plsc.subcore_barrier()

        def fire_gathers(slot):
            for j in range(NB):
                pltpu.async_copy(t_hbm.at[gidx_w.at[slot, j]],
                                 rows.at[slot, j], gsems[slot])

        def drain_gathers(slot):
            for j in range(NB):
                pltpu.make_async_copy(t_hbm.at[gidx_w.at[slot, j]],
                                      rows.at[slot, j], gsems[slot]).wait()

        def fire_scatters(slot):
            return [pltpu.async_copy(rows.at[slot, j],
                                     shared.at[col_w.at[slot, j]],
                                     ssems[slot], add=True)
                    for j in range(NB)]

        fire_gathers(0)
        fire_gathers(1)

        def body(kk, c):
            drain_gathers(0)
            spa = fire_scatters(0)
            drain_gathers(1)
            spb = fire_scatters(1)
            offa, offb = _batch_offsets(kk)
            for cp in spa:
                cp.wait()
            ia = [pltpu.async_copy(gidx_hbm.at[wid, pl.ds(offa, NB)],
                                   gidx_w.at[0], isem0),
                  pltpu.async_copy(col_hbm.at[wid, pl.ds(offa, NB)],
                                   col_w.at[0], isem0)]
            for cp in spb:
                cp.wait()
            ib = [pltpu.async_copy(gidx_hbm.at[wid, pl.ds(offb, NB)],
                                   gidx_w.at[1], isem1),
                  pltpu.async_copy(col_hbm.at[wid, pl.ds(offb, NB)],
                                   col_w.at[1], isem1)]
            for cp in ia:
                cp.wait()
            fire_gathers(0)
            for cp in ib:
                cp.wait()
            fire_gathers(1)
            return c
        lax.fori_loop(0, NPAIR, body, 0)

        drain_gathers(0)
        spa = fire_scatters(0)
        drain_gathers(1)
        for cp in spa:
            cp.wait()
        plsc.subcore_barrier()
        _copy_out(shared, out_hbm, ci, sid)

    return k(table, gidx, colp)


def _splitpad(a, fill):
    pad = jnp.full((EPAD - E,), fill, jnp.int32)
    return jnp.concatenate([a.astype(jnp.int32), pad]).reshape(NW, NCHUNK, CHUNK)


def kernel(var_node_features, con_node_features, edge_features, rhs, params,
           edge_index, edge_types, assoc_con, assoc_var, node_types):
    row = edge_index[0]
    col = edge_index[1]

    gidx = _splitpad(row + N * edge_types, 0)
    colp = _splitpad(col, N)
    rowp = _splitpad(row, N)
    ef = edge_features
    rhs2d = rhs[:, None]
    feats = jnp.concatenate([var_node_features, con_node_features], axis=0)
    feats = jnp.pad(feats, ((0, 0), (0, DIM - feats.shape[1])))
    ew1 = jnp.stack([_p16(params["var_mlp"]["l1"]["w"]),
                     _p16(params["con_mlp"]["l1"]["w"])])
    eb1 = jnp.stack([_pb16(params["var_mlp"]["l1"]["b"]),
                     _pb16(params["con_mlp"]["l1"]["b"])])
    ew2 = jnp.stack([_p16(params["var_mlp"]["l2"]["w"]),
                     _p16(params["con_mlp"]["l2"]["w"])])
    eb2 = jnp.stack([_pb16(params["var_mlp"]["l2"]["b"]),
                     _pb16(params["con_mlp"]["l2"]["b"])])
    convw = [_conv_weights(cp) for cp in params["convs"]]
    biases = [cp["bias"][None, :] for cp in params["convs"]]

    xsh = jax.ShapeDtypeStruct((N, DIM), jnp.float32)
    tsh = jax.ShapeDtypeStruct((2, N, DIM), jnp.float32)
    csh = jax.ShapeDtypeStruct((N, 1), jnp.float32)

    emb_spec = pl.BlockSpec((1, DIM, DIM), lambda i: (i // VBLK, 0, 0))
    emb_bspec = pl.BlockSpec((1, 1, DIM), lambda i: (i // VBLK, 0, 0))
    x0 = pl.pallas_call(
        _embed_body, grid=(NBLK,),
        in_specs=[_XBLK, emb_spec, emb_bspec, emb_spec, emb_bspec],
        out_specs=_XBLK, out_shape=xsh,
    )(feats, ew1, eb1, ew2, eb2)

    degp = _sc_degree(rowp)

    pspec = pl.BlockSpec((2, BLK, DIM), lambda i: (0, i, 0))
    t1, inv_deg = pl.pallas_call(
        _t1_body, grid=(NBLK,),
        in_specs=[_XBLK, pspec, _CBLK] + _CONV_W_SPECS,
        out_specs=(_TBLK, _CBLK), out_shape=(tsh, csh),
    )(x0, degp, ef, *convw[0])

    rhs_spec = pl.BlockSpec((BLK, 1), lambda i: (jnp.maximum(i - VBLK, 0), 0))
    xs = [x0]
    t = t1
    for layer in range(6):
        partial = _sc_aggregate(t.reshape(2 * N, DIM), gidx, colp)
        if layer < 5:
            x, t = pl.pallas_call(
                _post_body, grid=(NBLK,),
                in_specs=[pspec, rhs_spec, _B16, _CBLK, _CBLK] + _CONV_W_SPECS,
                out_specs=(_XBLK, _TBLK), out_shape=(xsh, tsh),
            )(partial, rhs2d, biases[layer], inv_deg, ef, *convw[layer + 1])
        else:
            x = pl.pallas_call(
                _post_final_body, grid=(NBLK,),
                in_specs=[pspec, rhs_spec, _B16],
                out_specs=_XBLK, out_shape=xsh,
            )(partial, rhs2d, biases[layer])
        xs.append(x)

    vspec = pl.BlockSpec((BLK, DIM), lambda i: (i, 0))
    w1spec = pl.BlockSpec((7 * DIM, DIM), lambda i: (0, 0))
    w4spec = pl.BlockSpec((DIM, 1), lambda i: (0, 0))
    b4spec = pl.BlockSpec((1, 1), lambda i: (0, 0))
    out = pl.pallas_call(
        _head_body, grid=(VBLK,),
        in_specs=[vspec] * 7 + [w1spec, _B16, _W16, _B16, _W16, _B16,
                                w4spec, b4spec],
        out_specs=pl.BlockSpec((BLK, 1), lambda i: (i, 0)),
        out_shape=jax.ShapeDtypeStruct((NV, 1), jnp.float32),
    )(*xs,
      params["fc1"]["w"], params["fc1"]["b"][None, :],
      params["fc2"]["w"], params["fc2"]["b"][None, :],
      params["fc3"]["w"], params["fc3"]["b"][None, :],
      params["fc4"]["w"], params["fc4"]["b"][None, :])
    return out[:, 0]


# speedup vs baseline: 59.7283x; 1.0091x over previous
import functools

import jax
import jax.numpy as jnp
from jax import lax
from jax.experimental import pallas as pl
from jax.experimental.pallas import tpu as pltpu
from jax.experimental.pallas import tpu_sc as plsc

DIM = 16
NV = 25000
N = 50000
E = 800000

NCORES = 2
NSUB = 16
NW = NCORES * NSUB
CHUNK = 128
NB = 8
NCHUNK = -(-E // (NW * CHUNK * NB)) * NB
NBATCH = NCHUNK // NB
NPAIR = NBATCH // 2
LASTB = (NBATCH - 1) * NB
EW = NCHUNK * CHUNK
EPAD = NW * EW

AGG_ROWS = 50048
ROWS_PER_TILE = AGG_ROWS // NSUB
ZROWS = 391

BLK = 5000
NBLK = N // BLK
VBLK = NV // BLK


def _p16(w):
    out = jnp.zeros((DIM, DIM), jnp.float32)
    return out.at[: w.shape[0], : w.shape[1]].set(w)


def _pb16(b):
    return jnp.zeros((1, DIM), jnp.float32).at[0, : b.shape[0]].set(b)


def _conv_weights(cp):
    wh2 = jnp.zeros((DIM, DIM), jnp.float32).at[:15, 15].set(
        cp["hidden_to_var"]["l2"]["w"][:, 0])
    bh2 = jnp.zeros((1, DIM), jnp.float32).at[0, 15].set(
        cp["hidden_to_var"]["l2"]["b"][0])
    return (
        _p16(cp["mlp_cons"]["l1"]["w"]), _pb16(cp["mlp_cons"]["l1"]["b"]),
        _p16(cp["mlp_cons"]["l2"]["w"]), _pb16(cp["mlp_cons"]["l2"]["b"]),
        _p16(cp["mlp_vars"]["l1"]["w"]), _pb16(cp["mlp_vars"]["l1"]["b"]),
        _p16(cp["mlp_vars"]["l2"]["w"]), _pb16(cp["mlp_vars"]["l2"]["b"]),
        _p16(cp["hidden_to_var"]["l1"]["w"]), _pb16(cp["hidden_to_var"]["l1"]["b"]),
        wh2, bh2,
    )


_W16 = pl.BlockSpec((DIM, DIM), lambda i: (0, 0))
_B16 = pl.BlockSpec((1, DIM), lambda i: (0, 0))
_XBLK = pl.BlockSpec((BLK, DIM), lambda i: (i, 0))
_CBLK = pl.BlockSpec((BLK, 1), lambda i: (i, 0))
_TBLK = pl.BlockSpec((2, BLK, DIM), lambda i: (0, i, 0))
_CONV_W_SPECS = [_W16, _B16] * 6


def _dot(a, b):
    return jnp.dot(a, b, preferred_element_type=jnp.float32)


def _col_mask(k):
    c = lax.broadcasted_iota(jnp.int32, (1, DIM), 1)
    return (c == k).astype(jnp.float32)


def _tables(x, idg, ef, wc1, bc1, wc2, bc2, wv1, bv1, wv2, bv2, wh1, bh1,
            wh2, bh2):
    hc = jnp.maximum(_dot(x, wc1) + bc1, 0.0)
    mc = _dot(hc, wc2) + bc2
    hv = jnp.maximum(_dot(x, wv1) + bv1, 0.0)
    mv = _dot(hv, wv2) + bv2
    hh = jnp.maximum(_dot(x, wh1) + bh1, 0.0)
    va = _dot(hh, wh2) + bh2
    m15 = _col_mask(15)
    m0 = mc * idg + va * ef
    m1 = (mv + (x[:, 15:16] * ef) * m15) * idg
    return m0, m1


def _embed_body(f_ref, w1_ref, b1_ref, w2_ref, b2_ref, o_ref):
    f = f_ref[...]
    h = jnp.maximum(_dot(f, w1_ref[0]) + b1_ref[0], 0.0)
    y = _dot(h, w2_ref[0]) + b2_ref[0]
    y = y + f[:, 0:1] * _col_mask(13) + f[:, 1:2] * _col_mask(14)
    o_ref[...] = y


def _t1_body(x_ref, degp_ref, ef_ref, *refs):
    (wc1, bc1, wc2, bc2, wv1, bv1, wv2, bv2, wh1, bh1, wh2, bh2,
     t_ref, idg_ref) = refs
    deg = degp_ref[0, :, 0:1] + degp_ref[1, :, 0:1]
    idg = jnp.where(deg > 0, 1.0 / jnp.maximum(deg, 1.0), 0.0)
    idg_ref[...] = idg
    m0, m1 = _tables(x_ref[...], idg, ef_ref[...],
                     wc1[...], bc1[0], wc2[...], bc2[0],
                     wv1[...], bv1[0], wv2[...], bv2[0],
                     wh1[...], bh1[0], wh2[...], bh2[0])
    t_ref[0] = m0
    t_ref[1] = m1


def _post_epilogue(p_ref, rhs_ref, bias_ref):
    i = pl.program_id(0)
    aggr = p_ref[0] + p_ref[1]
    con = jnp.where(i >= VBLK, 1.0, 0.0)
    aggr = aggr - rhs_ref[...] * _col_mask(15) * con
    return jnp.maximum(aggr + bias_ref[0], 0.0)


def _post_body(p_ref, rhs_ref, bias_ref, idg_ref, ef_ref, *refs):
    (wc1, bc1, wc2, bc2, wv1, bv1, wv2, bv2, wh1, bh1, wh2, bh2,
     x_ref, t_ref) = refs
    x = _post_epilogue(p_ref, rhs_ref, bias_ref)
    x_ref[...] = x
    m0, m1 = _tables(x, idg_ref[...], ef_ref[...],
                     wc1[...], bc1[0], wc2[...], bc2[0],
                     wv1[...], bv1[0], wv2[...], bv2[0],
                     wh1[...], bh1[0], wh2[...], bh2[0])
    t_ref[0] = m0
    t_ref[1] = m1


def _post_final_body(p_ref, rhs_ref, bias_ref, x_ref):
    x_ref[...] = _post_epilogue(p_ref, rhs_ref, bias_ref)


def _head_body(*refs):
    xs = refs[0:7]
    w1, b1, w2, b2, w3, b3, w4, b4, o_ref = refs[7:]
    w1v = w1[...]
    acc = b1[0]
    for k in range(7):
        acc = acc + _dot(xs[k][...], w1v[16 * k:16 * (k + 1)])
    h = jnp.maximum(acc, 0.0)
    h = jnp.maximum(_dot(h, w2[...]) + b2[0], 0.0)
    h = jnp.maximum(_dot(h, w3[...]) + b3[0], 0.0)
    o_ref[...] = _dot(h, w4[...]) + b4[...]


def _sc_mesh():
    return plsc.VectorSubcoreMesh(core_axis_name="c", subcore_axis_name="s",
                                  num_cores=NCORES, num_subcores=NSUB)


_SC_PARAMS = pltpu.CompilerParams(use_tc_tiling_on_sc=False)


def _fill_rows(ref, nrows, value):
    def body(i, c):
        ref[i] = jnp.full((DIM,), value, jnp.float32)
        return c
    lax.fori_loop(0, nrows, body, 0)


def _zero_shared_async(shared, zbuf, sid, zsem):
    _fill_rows(zbuf, ZROWS, 0.0)
    base = sid * ROWS_PER_TILE
    return [pltpu.async_copy(zbuf, shared.at[pl.ds(base + t * ZROWS, ZROWS)],
                             zsem)
            for t in range(ROWS_PER_TILE // ZROWS)]


def _copy_out(shared, out_hbm, ci, sid):
    base = sid * ROWS_PER_TILE
    pltpu.sync_copy(shared.at[pl.ds(base, ROWS_PER_TILE)],
                    out_hbm.at[ci, pl.ds(base, ROWS_PER_TILE)])


def _batch_offsets(kk):
    offa = pl.multiple_of(jnp.minimum((2 * kk + 2) * NB, LASTB), NB)
    offb = pl.multiple_of(jnp.minimum((2 * kk + 3) * NB, LASTB), NB)
    return offa, offb


def _sc_degree(rowp):
    @functools.partial(
        pl.kernel,
        out_type=jax.ShapeDtypeStruct((NCORES, AGG_ROWS, DIM), jnp.float32),
        mesh=_sc_mesh(),
        compiler_params=_SC_PARAMS,
        scratch_types=[
            pltpu.VMEM_SHARED((AGG_ROWS, DIM), jnp.float32),
            pltpu.VMEM((2, NB, CHUNK), jnp.int32),
            pltpu.VMEM((CHUNK, DIM), jnp.float32),
            pltpu.VMEM((ZROWS, DIM), jnp.float32),
            pltpu.SemaphoreType.DMA,
            pltpu.SemaphoreType.DMA,
            pltpu.SemaphoreType.DMA,
            pltpu.SemaphoreType.DMA,
            pltpu.SemaphoreType.DMA,
        ],
    )
    def k(row_hbm, out_hbm, shared, idx_w, ones_v, zbuf,
          ssem0, ssem1, isem0, isem1, zsem):
        ci = lax.axis_index("c")
        sid = lax.axis_index("s")
        wid = sid * NCORES + ci
        zcps = _zero_shared_async(shared, zbuf, sid, zsem)
        pltpu.sync_copy(row_hbm.at[wid, pl.ds(0, NB)], idx_w.at[0])
        pltpu.sync_copy(row_hbm.at[wid, pl.ds(NB, NB)], idx_w.at[1])
        _fill_rows(ones_v, CHUNK, 1.0)
        for cp in zcps:
            cp.wait()
        plsc.subcore_barrier()

        def fire(slot, sem):
            return [pltpu.async_copy(ones_v, shared.at[idx_w.at[slot, j]],
                                     sem, add=True)
                    for j in range(NB)]

        def body(kk, c):
            spa = fire(0, ssem0)
            spb = fire(1, ssem1)
            offa, offb = _batch_offsets(kk)
            for cp in spa:
                cp.wait()
            ia = pltpu.async_copy(row_hbm.at[wid, pl.ds(offa, NB)],
                                  idx_w.at[0], isem0)
            for cp in spb:
                cp.wait()
            ib = pltpu.async_copy(row_hbm.at[wid, pl.ds(offb, NB)],
                                  idx_w.at[1], isem1)
            ia.wait()
            ib.wait()
            return c
        lax.fori_loop(0, NPAIR, body, 0)

        for cp in fire(0, ssem0):
            cp.wait()
        plsc.subcore_barrier()
        _copy_out(shared, out_hbm, ci, sid)

    return k(rowp)


def _sc_aggregate(table, gidx, colp):
    @functools.partial(
        pl.kernel,
        out_type=jax.ShapeDtypeStruct((NCORES, AGG_ROWS, DIM), jnp.float32),
        mesh=_sc_mesh(),
        compiler_params=_SC_PARAMS,
        scratch_types=[
            pltpu.VMEM_SHARED((AGG_ROWS, DIM), jnp.float32),
            pltpu.VMEM((2, NB, CHUNK), jnp.int32),
            pltpu.VMEM((2, NB, CHUNK), jnp.int32),
            pltpu.VMEM((2, NB, CHUNK, DIM), jnp.float32),
            pltpu.VMEM((ZROWS, DIM), jnp.float32),
            pltpu.SemaphoreType.DMA,
            pltpu.SemaphoreType.DMA,
            pltpu.SemaphoreType.DMA,
            pltpu.SemaphoreType.DMA,
            pltpu.SemaphoreType.DMA,
            pltpu.SemaphoreType.DMA,
            pltpu.SemaphoreType.DMA,
        ],
    )
    def k(t_hbm, gidx_hbm, col_hbm, out_hbm, shared, gidx_w, col_w, rows,
          zbuf, gsem0, gsem1, ssem0, ssem1, isem0, isem1, zsem):
        ci = lax.axis_index("c")
        sid = lax.axis_index("s")
        wid = sid * NCORES + ci
        gsems = (gsem0, gsem1)
        ssems = (ssem0, ssem1)
        zcps = _zero_shared_async(shared, zbuf, sid, zsem)
        pltpu.sync_copy(gidx_hbm.at[wid, pl.ds(0, NB)], gidx_w.at[0])
        pltpu.sync_copy(col_hbm.at[wid, pl.ds(0, NB)], col_w.at[0])
        pltpu.sync_copy(gidx_hbm.at[wid, pl.ds(NB, NB)], gidx_w.at[1])
        pltpu.sync_copy(col_hbm.at[wid, pl.ds(NB, NB)], col_w.at[1])
        for cp in zcps:
            cp.wait()
        plsc.subcore_barrier()

        def fire_gathers(slot):
            for j in range(NB):
                pltpu.async_copy(t_hbm.at[gidx_w.at[slot, j]],
                                 rows.at[slot, j], gsems[slot])

        def drain_gathers(slot):
            for j in range(NB):
                pltpu.make_async_copy(t_hbm.at[gidx_w.at[slot, j]],
                                      rows.at[slot, j], gsems[slot]).wait()

        def fire_scatters(slot):
            return [pltpu.async_copy(rows.at[slot, j],
                                     shared.at[col_w.at[slot, j]],
                                     ssems[slot], add=True)
                    for j in range(NB)]

        fire_gathers(0)
        fire_gathers(1)

        def body(kk, c):
            drain_gathers(0)
            spa = fire_scatters(0)
            drain_gathers(1)
            spb = fire_scatters(1)
            offa, offb = _batch_offsets(kk)
            for cp in spa:
                cp.wait()
            ia = [pltpu.async_copy(gidx_hbm.at[wid, pl.ds(offa, NB)],
                                   gidx_w.at[0], isem0),
                  pltpu.async_copy(col_hbm.at[wid, pl.ds(offa, NB)],
                                   col_w.at[0], isem0)]
            for cp in spb:
                cp.wait()
            ib = [pltpu.async_copy(gidx_hbm.at[wid, pl.ds(offb, NB)],
                                   gidx_w.at[1], isem1),
                  pltpu.async_copy(col_hbm.at[wid, pl.ds(offb, NB)],
                                   col_w.at[1], isem1)]
            for cp in ia:
                cp.wait()
            fire_gathers(0)
            for cp in ib:
                cp.wait()
            fire_gathers(1)
            return c
        lax.fori_loop(0, NPAIR, body, 0)

        drain_gathers(0)
        spa = fire_scatters(0)
        drain_gathers(1)
        for cp in spa:
            cp.wait()
        plsc.subcore_barrier()
        _copy_out(shared, out_hbm, ci, sid)

    return k(table, gidx, colp)


def _splitpad(a, fill):
    pad = jnp.full((EPAD - E,), fill, jnp.int32)
    return jnp.concatenate([a.astype(jnp.int32), pad]).reshape(NW, NCHUNK, CHUNK)


def kernel(var_node_features, con_node_features, edge_features, rhs, params,
           edge_index, edge_types, assoc_con, assoc_var, node_types):
    row = edge_index[0]
    col = edge_index[1]

    gidx = _splitpad(row + N * edge_types, 0)
    colp = _splitpad(col, N)
    rowp = _splitpad(row, N)
    ef = edge_features
    rhs2d = rhs[:, None]
    feats = jnp.concatenate([var_node_features, con_node_features], axis=0)
    feats = jnp.pad(feats, ((0, 0), (0, DIM - feats.shape[1])))
    ew1 = jnp.stack([_p16(params["var_mlp"]["l1"]["w"]),
                     _p16(params["con_mlp"]["l1"]["w"])])
    eb1 = jnp.stack([_pb16(params["var_mlp"]["l1"]["b"]),
                     _pb16(params["con_mlp"]["l1"]["b"])])
    ew2 = jnp.stack([_p16(params["var_mlp"]["l2"]["w"]),
                     _p16(params["con_mlp"]["l2"]["w"])])
    eb2 = jnp.stack([_pb16(params["var_mlp"]["l2"]["b"]),
                     _pb16(params["con_mlp"]["l2"]["b"])])
    convw = [_conv_weights(cp) for cp in params["convs"]]
    biases = [cp["bias"][None, :] for cp in params["convs"]]

    xsh = jax.ShapeDtypeStruct((N, DIM), jnp.float32)
    tsh = jax.ShapeDtypeStruct((2, N, DIM), jnp.float32)
    csh = jax.ShapeDtypeStruct((N, 1), jnp.float32)

    emb_spec = pl.BlockSpec((1, DIM, DIM), lambda i: (i // VBLK, 0, 0))
    emb_bspec = pl.BlockSpec((1, 1, DIM), lambda i: (i // VBLK, 0, 0))
    x0 = pl.pallas_call(
        _embed_body, grid=(NBLK,),
        in_specs=[_XBLK, emb_spec, emb_bspec, emb_spec, emb_bspec],
        out_specs=_XBLK, out_shape=xsh,
    )(feats, ew1, eb1, ew2, eb2)

    degp = _sc_degree(rowp)

    pspec = pl.BlockSpec((2, BLK, DIM), lambda i: (0, i, 0))
    t1, inv_deg = pl.pallas_call(
        _t1_body, grid=(NBLK,),
        in_specs=[_XBLK, pspec, _CBLK] + _CONV_W_SPECS,
        out_specs=(_TBLK, _CBLK), out_shape=(tsh, csh),
    )(x0, degp, ef, *convw[0])

    rhs_spec = pl.BlockSpec((BLK, 1), lambda i: (jnp.maximum(i - VBLK, 0), 0))
    xv_spec = pl.BlockSpec((BLK, DIM), lambda i: (jnp.minimum(i, VBLK), 0))
    xvsh = jax.ShapeDtypeStruct((NV + BLK, DIM), jnp.float32)
    xs = [x0]
    t = t1
    for layer in range(6):
        partial = _sc_aggregate(t.reshape(2 * N, DIM), gidx, colp)
        if layer < 5:
            x, t = pl.pallas_call(
                _post_body, grid=(NBLK,),
                in_specs=[pspec, rhs_spec, _B16, _CBLK, _CBLK] + _CONV_W_SPECS,
                out_specs=(xv_spec, _TBLK), out_shape=(xvsh, tsh),
            )(partial, rhs2d, biases[layer], inv_deg, ef, *convw[layer + 1])
        else:
            x = pl.pallas_call(
                _post_final_body, grid=(NBLK,),
                in_specs=[pspec, rhs_spec, _B16],
                out_specs=xv_spec, out_shape=xvsh,
            )(partial, rhs2d, biases[layer])
        xs.append(x)

    vspec = pl.BlockSpec((BLK, DIM), lambda i: (i, 0))
    w1spec = pl.BlockSpec((7 * DIM, DIM), lambda i: (0, 0))
    w4spec = pl.BlockSpec((DIM, 1), lambda i: (0, 0))
    b4spec = pl.BlockSpec((1, 1), lambda i: (0, 0))
    out = pl.pallas_call(
        _head_body, grid=(VBLK,),
        in_specs=[vspec] * 7 + [w1spec, _B16, _W16, _B16, _W16, _B16,
                                w4spec, b4spec],
        out_specs=pl.BlockSpec((BLK, 1), lambda i: (i, 0)),
        out_shape=jax.ShapeDtypeStruct((NV, 1), jnp.float32),
    )(*xs,
      params["fc1"]["w"], params["fc1"]["b"][None, :],
      params["fc2"]["w"], params["fc2"]["b"][None, :],
      params["fc3"]["w"], params["fc3"]["b"][None, :],
      params["fc4"]["w"], params["fc4"]["b"][None, :])
    return out[:, 0]
